# baseline (speedup 1.0000x reference)
import functools

import jax
import jax.numpy as jnp
from jax import lax
from jax.experimental import pallas as pl
from jax.experimental.pallas import tpu as pltpu
from jax.experimental.pallas import tpu_sc as plsc

N = 10000
NPAD = 10240
E = 320000
EPAD = 327680
D = 128
G = 64
NI = 3

NC = 2
NS = 16
NW = NC * NS

EW = EPAD // NW
ECH = 128
NCH = EW // ECH
NSL = NPAD // NS
ECH3 = 32
NCH3 = EW // ECH3
KB = 8

_mesh = plsc.VectorSubcoreMesh(core_axis_name="c", subcore_axis_name="s")


@functools.partial(
    pl.kernel,
    out_type=jax.ShapeDtypeStruct((NC * NPAD,), jnp.float32),
    mesh=_mesh,
    compiler_params=pltpu.CompilerParams(needs_layout_passes=False),
    scratch_types=[
        pltpu.VMEM((NCH, ECH), jnp.int32),
        pltpu.VMEM((ECH,), jnp.float32),
        pltpu.VMEM((NSL,), jnp.float32),
        pltpu.VMEM_SHARED((NPAD,), jnp.float32),
    ],
)
def _k1_deg(dst_r, ones128, zflat, out_hbm, dsti, ones_v, zv, deg_sh):
    c = lax.axis_index("c")
    s = lax.axis_index("s")
    w = s * NC + c
    pltpu.sync_copy(zflat.at[pl.ds(s * NSL, NSL)], zv)
    pltpu.sync_copy(zv, deg_sh.at[pl.ds(s * NSL, NSL)])
    pltpu.sync_copy(ones128, ones_v)
    pltpu.sync_copy(dst_r.at[pl.ds(w * NCH, NCH)], dsti)
    plsc.subcore_barrier()

    @pl.loop(0, NCH)
    def _(j):
        pltpu.sync_copy(ones_v, deg_sh.at[dsti.at[j]], add=True)

    plsc.subcore_barrier()
    pltpu.sync_copy(deg_sh.at[pl.ds(s * NSL, NSL)], zv)
    pltpu.sync_copy(zv, out_hbm.at[pl.ds(c * NPAD + s * NSL, NSL)])


P3 = 8
PCH = NCH3 // P3


@functools.partial(
    pl.kernel,
    out_type=jax.ShapeDtypeStruct((NC * NPAD, D), jnp.float32),
    mesh=_mesh,
    compiler_params=pltpu.CompilerParams(needs_layout_passes=False),
    scratch_types=[
        pltpu.VMEM((PCH, ECH3), jnp.int32),
        pltpu.VMEM((PCH, ECH3), jnp.int32),
        pltpu.VMEM((KB, ECH3, D), jnp.float32),
        pltpu.VMEM_SHARED((NPAD, D), jnp.float32),
        pltpu.SemaphoreType.DMA((KB,)),
        pltpu.SemaphoreType.DMA((KB,)),
    ],
)
def _k3_edge(src_r, dst_r, xs_hbm, zrows, out_hbm,
             srci, dsti, rows, acc_sh, gsem, ssem):
    c = lax.axis_index("c")
    s = lax.axis_index("s")
    w = s * NC + c
    pltpu.sync_copy(zrows, rows.at[0])

    @pl.loop(0, NSL // ECH3)
    def _(j):
        pltpu.sync_copy(
            rows.at[0], acc_sh.at[pl.ds(s * NSL + j * ECH3, ECH3)])

    plsc.subcore_barrier()

    @pl.loop(0, P3)
    def _(p):
        pltpu.sync_copy(src_r.at[pl.ds(w * NCH3 + p * PCH, PCH)], srci)
        pltpu.sync_copy(dst_r.at[pl.ds(w * NCH3 + p * PCH, PCH)], dsti)

        @pl.loop(0, PCH // KB)
        def _(m):
            @pl.loop(0, KB)
            def _(b):
                j = m * KB + b

                @pl.when(m > 0)
                def _():
                    pltpu.make_async_copy(
                        rows.at[b], acc_sh.at[dsti.at[j - KB]],
                        ssem.at[b]).wait()

                pltpu.async_copy(xs_hbm.at[srci.at[j]], rows.at[b], gsem.at[b])

            @pl.loop(0, KB)
            def _(b):
                j = m * KB + b
                pltpu.make_async_copy(
                    xs_hbm.at[srci.at[j]], rows.at[b], gsem.at[b]).wait()
                pltpu.async_copy(
                    rows.at[b], acc_sh.at[dsti.at[j]], ssem.at[b], add=True)

        @pl.loop(0, KB)
        def _(b):
            j = PCH - KB + b
            pltpu.make_async_copy(
                rows.at[b], acc_sh.at[dsti.at[j]], ssem.at[b]).wait()

    plsc.subcore_barrier()

    @pl.loop(0, NSL // ECH3)
    def _(j):
        pltpu.sync_copy(acc_sh.at[pl.ds(s * NSL + j * ECH3, ECH3)], rows.at[0])
        pltpu.sync_copy(
            rows.at[0], out_hbm.at[pl.ds(c * NPAD + s * NSL + j * ECH3, ECH3)])


@functools.partial(
    pl.kernel,
    out_type=jax.ShapeDtypeStruct((NC * NI * NPAD,), jnp.float32),
    mesh=_mesh,
    compiler_params=pltpu.CompilerParams(needs_layout_passes=False),
    scratch_types=[
        pltpu.VMEM((EW,), jnp.int32),
        pltpu.VMEM((EW,), jnp.int32),
        pltpu.VMEM((NI * NPAD,), jnp.float32),
        pltpu.VMEM((ECH,), jnp.float32),
        pltpu.VMEM((ECH,), jnp.int32),
        pltpu.VMEM((NI * NSL,), jnp.float32),
        pltpu.VMEM_SHARED((NI * NPAD,), jnp.float32),
    ],
)
def _k5_edge_t(src_f, dst_f, t3f, zflat3, out_hbm,
               srci, dsti, tv, vals, dsto, zv, agg_sh):
    c = lax.axis_index("c")
    s = lax.axis_index("s")
    w = s * NC + c
    pltpu.sync_copy(zflat3.at[pl.ds(s * NI * NSL, NI * NSL)], zv)
    pltpu.sync_copy(zv, agg_sh.at[pl.ds(s * NI * NSL, NI * NSL)])
    pltpu.sync_copy(t3f, tv)
    pltpu.sync_copy(src_f.at[pl.ds(w * EW, EW)], srci)
    pltpu.sync_copy(dst_f.at[pl.ds(w * EW, EW)], dsti)
    plsc.subcore_barrier()

    for i in range(NI):
        @pl.loop(0, NCH)
        def _(j):
            @pl.loop(0, ECH // 16)
            def _(k):
                sv = srci[pl.ds(j * ECH + k * 16, 16)] + i * NPAD
                vals[pl.ds(k * 16, 16)] = plsc.load_gather(tv, [sv])
                dsto[pl.ds(k * 16, 16)] = (
                    dsti[pl.ds(j * ECH + k * 16, 16)] + i * NPAD)

            pltpu.sync_copy(vals, agg_sh.at[dsto], add=True)

    plsc.subcore_barrier()
    pltpu.sync_copy(agg_sh.at[pl.ds(s * NI * NSL, NI * NSL)], zv)
    pltpu.sync_copy(
        zv, out_hbm.at[pl.ds(c * NI * NPAD + s * NI * NSL, NI * NSL)])


def _k2_body(degp_ref, x_ref, xs_ref):
    degp = degp_ref[...]
    deg = degp[0] + degp[1] + 1.0
    dinv = lax.rsqrt(deg)
    xs_ref[...] = x_ref[...] * dinv


def _k2_scale(degp, x_pad):
    B = 1024
    grid = NPAD // B
    return pl.pallas_call(
        _k2_body,
        grid=(grid,),
        in_specs=[
            pl.BlockSpec((NC, B, 1), lambda i: (0, i, 0)),
            pl.BlockSpec((B, D), lambda i: (i, 0)),
        ],
        out_specs=pl.BlockSpec((B, D), lambda i: (i, 0)),
        out_shape=jax.ShapeDtypeStruct((NPAD, D), jnp.float32),
    )(degp, x_pad)


def _k4_body(degp_ref, accp_ref, x_ref, wg_ref, bg_ref, wrr_ref,
             g_ref, t_ref, r_ref):
    degp = degp_ref[...]
    deg = degp[0] + degp[1] + 1.0
    dinv = lax.rsqrt(deg)
    accp = accp_ref[...]
    y = dinv * (accp[0] + accp[1]) + (dinv * dinv) * x_ref[...]
    wrr = wrr_ref[...]
    for i in range(NI):
        gi = jnp.maximum(
            jax.lax.dot_general(
                y, wg_ref[i], (((1,), (0,)), ((), ())),
                precision=lax.Precision.HIGHEST,
                preferred_element_type=jnp.float32) + bg_ref[i], 0.0)
        g_ref[i] = gi
        tr = jax.lax.dot_general(
            gi, wrr, (((1,), (0,)), ((), ())),
            precision=lax.Precision.HIGHEST,
            preferred_element_type=jnp.float32)
        t_ref[i] = tr[:, 0]
        r_ref[i] = tr[:, 1]


def _k4_layers(degp, accp, x_pad, W_gcn, b_gcn, Wrr):
    B = 1024
    grid = NPAD // B
    return pl.pallas_call(
        _k4_body,
        grid=(grid,),
        in_specs=[
            pl.BlockSpec((NC, B, 1), lambda i: (0, i, 0)),
            pl.BlockSpec((NC, B, D), lambda i: (0, i, 0)),
            pl.BlockSpec((B, D), lambda i: (i, 0)),
            pl.BlockSpec((NI, D, D), lambda i: (0, 0, 0)),
            pl.BlockSpec((NI, D), lambda i: (0, 0)),
            pl.BlockSpec((D, 2), lambda i: (0, 0)),
        ],
        out_specs=[
            pl.BlockSpec((NI, B, D), lambda i: (0, i, 0)),
            pl.BlockSpec((NI, B), lambda i: (0, i)),
            pl.BlockSpec((NI, B), lambda i: (0, i)),
        ],
        out_shape=[
            jax.ShapeDtypeStruct((NI, NPAD, D), jnp.float32),
            jax.ShapeDtypeStruct((NI, NPAD), jnp.float32),
            jax.ShapeDtypeStruct((NI, NPAD), jnp.float32),
        ],
    )(degp, accp, x_pad, W_gcn, b_gcn, Wrr)


def _k6_body(g_ref, aggp_ref, r_ref, batch_ref, wgo_ref, bgo_ref,
             a2_ref, ab_ref, batt_ref, out_ref):
    batch = batch_ref[...]
    gid = lax.broadcasted_iota(jnp.int32, (G, NPAD), 0)
    M = batch == gid
    Mf = M.astype(jnp.float32)
    aggp = aggp_ref[...]
    agg = aggp[0] + aggp[1]
    r3 = r_ref[...]
    batt = batt_ref[...]
    wgo = wgo_ref[...]
    bgo = bgo_ref[...]
    a2 = a2_ref[...]
    ab = ab_ref[...]

    neg = jnp.float32(-1e30)
    sc_cols = []
    for i in range(NI):
        xconv = (agg[i] + r3[i])[None, :] + batt
        m = jnp.max(jnp.where(M, xconv, neg), axis=1, keepdims=True)
        mx = jnp.max(jnp.where(M, m, neg), axis=0, keepdims=True)
        e = jnp.exp(jnp.minimum(xconv - mx, 0.0))
        P = Mf * e
        den = jnp.sum(P, axis=1, keepdims=True)
        S = P / (den + 1e-16)
        gx = jax.lax.dot_general(
            S, g_ref[i], (((1,), (0,)), ((), ())),
            precision=lax.Precision.HIGHEST,
            preferred_element_type=jnp.float32)
        gout = jnp.tanh(
            jax.lax.dot_general(
                gx, wgo, (((1,), (0,)), ((), ())),
                precision=lax.Precision.HIGHEST,
                preferred_element_type=jnp.float32) + bgo)
        sc_i = jnp.sum(gout * a2[:, i][None, :], axis=1, keepdims=True)
        sc_cols.append(sc_i + ab[0:1, i:i + 1])
    alls = jnp.concatenate(sc_cols, axis=1)
    mm = jnp.max(alls, axis=1, keepdims=True)
    ee = jnp.exp(alls - mm)
    alpha = ee / jnp.sum(ee, axis=1, keepdims=True)
    rep = jax.lax.dot_general(
        Mf, alpha, (((0,), (0,)), ((), ())),
        precision=lax.Precision.HIGHEST,
        preferred_element_type=jnp.float32)
    total = g_ref[0] * rep[:, 0][:, None]
    for i in range(1, NI):
        total = total + g_ref[i] * rep[:, i][:, None]
    out_ref[...] = total[0:N, :]


def _k6_final(g, aggp, r3, batch2, W_gout, b_gout, a2, ab, batt):
    return pl.pallas_call(
        _k6_body,
        out_shape=jax.ShapeDtypeStruct((N, D), jnp.float32),
    )(g, aggp, r3, batch2, W_gout, b_gout, a2, ab, batt)


def kernel(x, edge_index, batch, W_gcn, b_gcn, W_rel, W_root, b_att,
           W_gout, b_gout, a, a_bias):
    f32 = jnp.float32
    src = edge_index[0].astype(jnp.int32)
    dst = edge_index[1].astype(jnp.int32)
    pad_i = jnp.full((EPAD - E,), N, dtype=jnp.int32)
    src_p = jnp.concatenate([src, pad_i])
    dst_p = jnp.concatenate([dst, pad_i])
    src_r = src_p.reshape(EPAD // ECH, ECH)
    dst_r = dst_p.reshape(EPAD // ECH, ECH)
    src_r3 = src_p.reshape(EPAD // ECH3, ECH3)
    dst_r3 = dst_p.reshape(EPAD // ECH3, ECH3)

    x_pad = jnp.concatenate(
        [x.astype(f32), jnp.zeros((NPAD - N, D), f32)], axis=0)
    batch2 = jnp.concatenate(
        [batch.astype(jnp.int32), jnp.full((NPAD - N,), G, jnp.int32)]
    ).reshape(1, NPAD)

    ones128 = jnp.ones((ECH,), f32)
    zflat = jnp.zeros((NPAD,), f32)
    zrows = jnp.zeros((ECH3, D), f32)
    zflat3 = jnp.zeros((NI * NPAD,), f32)

    degf = _k1_deg(dst_r, ones128, zflat)
    degp = degf.reshape(NC, NPAD, 1)
    xs = _k2_scale(degp, x_pad)

    accf = _k3_edge(src_r3, dst_r3, xs, zrows)
    accp = accf.reshape(NC, NPAD, D)

    Wrr = jnp.concatenate([W_rel.astype(f32), W_root.astype(f32)], axis=1)
    g, t3, r3 = _k4_layers(degp, accp, x_pad, W_gcn.astype(f32),
                           b_gcn.astype(f32), Wrr)

    aggf = _k5_edge_t(src_p, dst_p, t3.reshape(NI * NPAD), zflat3)
    aggp = aggf.reshape(NC, NI, NPAD)

    a2 = a.astype(f32)[0]
    ab = a_bias.astype(f32)[0]
    batt = b_att.astype(f32).reshape(1, 1)
    out = _k6_final(g, aggp, r3, batch2, W_gout.astype(f32),
                    b_gout.astype(f32).reshape(1, D), a2, ab, batt)
    return out


# speedup vs baseline: 1.3149x; 1.3149x over previous
import functools

import jax
import jax.numpy as jnp
from jax import lax
from jax.experimental import pallas as pl
from jax.experimental.pallas import tpu as pltpu
from jax.experimental.pallas import tpu_sc as plsc

N = 10000
NPAD = 10240
E = 320000
EPAD = 327680
D = 128
G = 64
NI = 3

NC = 2
NS = 16
NW = NC * NS

EW = EPAD // NW
ECH = 128
NCH = EW // ECH
NSL = NPAD // NS
ECH3 = 64
NCH3 = EW // ECH3
KB = 4

_mesh = plsc.VectorSubcoreMesh(core_axis_name="c", subcore_axis_name="s")


@functools.partial(
    pl.kernel,
    out_type=jax.ShapeDtypeStruct((NC * NPAD,), jnp.float32),
    mesh=_mesh,
    compiler_params=pltpu.CompilerParams(needs_layout_passes=False),
    scratch_types=[
        pltpu.VMEM((NCH, ECH), jnp.int32),
        pltpu.VMEM((ECH,), jnp.float32),
        pltpu.VMEM((NSL,), jnp.float32),
        pltpu.VMEM_SHARED((NPAD,), jnp.float32),
    ],
)
def _k1_deg(dst_r, ones128, zflat, out_hbm, dsti, ones_v, zv, deg_sh):
    c = lax.axis_index("c")
    s = lax.axis_index("s")
    w = s * NC + c
    pltpu.sync_copy(zflat.at[pl.ds(s * NSL, NSL)], zv)
    pltpu.sync_copy(zv, deg_sh.at[pl.ds(s * NSL, NSL)])
    pltpu.sync_copy(ones128, ones_v)
    pltpu.sync_copy(dst_r.at[pl.ds(w * NCH, NCH)], dsti)
    plsc.subcore_barrier()

    @pl.loop(0, NCH)
    def _(j):
        pltpu.sync_copy(ones_v, deg_sh.at[dsti.at[j]], add=True)

    plsc.subcore_barrier()
    pltpu.sync_copy(deg_sh.at[pl.ds(s * NSL, NSL)], zv)
    pltpu.sync_copy(zv, out_hbm.at[pl.ds(c * NPAD + s * NSL, NSL)])


P3 = 4
PCH = NCH3 // P3


@functools.partial(
    pl.kernel,
    out_type=jax.ShapeDtypeStruct((NC * NPAD, D), jnp.float32),
    mesh=_mesh,
    compiler_params=pltpu.CompilerParams(needs_layout_passes=False),
    scratch_types=[
        pltpu.VMEM((PCH, ECH3), jnp.int32),
        pltpu.VMEM((PCH, ECH3), jnp.int32),
        pltpu.VMEM((KB, ECH3, D), jnp.float32),
        pltpu.VMEM_SHARED((NPAD, D), jnp.float32),
        pltpu.SemaphoreType.DMA((KB,)),
        pltpu.SemaphoreType.DMA((KB,)),
    ],
)
def _k3_edge(src_r, dst_r, xs_hbm, zrows, out_hbm,
             srci, dsti, rows, acc_sh, gsem, ssem):
    c = lax.axis_index("c")
    s = lax.axis_index("s")
    w = s * NC + c
    pltpu.sync_copy(zrows, rows.at[0])

    @pl.loop(0, NSL // ECH3)
    def _(j):
        pltpu.sync_copy(
            rows.at[0], acc_sh.at[pl.ds(s * NSL + j * ECH3, ECH3)])

    plsc.subcore_barrier()

    @pl.loop(0, P3)
    def _(p):
        pltpu.sync_copy(src_r.at[pl.ds(w * NCH3 + p * PCH, PCH)], srci)
        pltpu.sync_copy(dst_r.at[pl.ds(w * NCH3 + p * PCH, PCH)], dsti)

        @pl.loop(0, PCH // KB)
        def _(m):
            @pl.loop(0, KB)
            def _(b):
                j = m * KB + b

                @pl.when(m > 0)
                def _():
                    pltpu.make_async_copy(
                        rows.at[b], acc_sh.at[dsti.at[j - KB]],
                        ssem.at[b]).wait()

                pltpu.async_copy(xs_hbm.at[srci.at[j]], rows.at[b], gsem.at[b])

            @pl.loop(0, KB)
            def _(b):
                j = m * KB + b
                pltpu.make_async_copy(
                    xs_hbm.at[srci.at[j]], rows.at[b], gsem.at[b]).wait()
                pltpu.async_copy(
                    rows.at[b], acc_sh.at[dsti.at[j]], ssem.at[b], add=True)

        @pl.loop(0, KB)
        def _(b):
            j = PCH - KB + b
            pltpu.make_async_copy(
                rows.at[b], acc_sh.at[dsti.at[j]], ssem.at[b]).wait()

    plsc.subcore_barrier()

    @pl.loop(0, NSL // ECH3)
    def _(j):
        pltpu.sync_copy(acc_sh.at[pl.ds(s * NSL + j * ECH3, ECH3)], rows.at[0])
        pltpu.sync_copy(
            rows.at[0], out_hbm.at[pl.ds(c * NPAD + s * NSL + j * ECH3, ECH3)])


@functools.partial(
    pl.kernel,
    out_type=jax.ShapeDtypeStruct((NC * NI * NPAD,), jnp.float32),
    mesh=_mesh,
    compiler_params=pltpu.CompilerParams(needs_layout_passes=False),
    scratch_types=[
        pltpu.VMEM((EW,), jnp.int32),
        pltpu.VMEM((EW,), jnp.int32),
        pltpu.VMEM((NI * NPAD,), jnp.float32),
        pltpu.VMEM((ECH,), jnp.float32),
        pltpu.VMEM((ECH,), jnp.int32),
        pltpu.VMEM((NI * NSL,), jnp.float32),
        pltpu.VMEM_SHARED((NI * NPAD,), jnp.float32),
    ],
)
def _k5_edge_t(src_f, dst_f, t3f, zflat3, out_hbm,
               srci, dsti, tv, vals, dsto, zv, agg_sh):
    c = lax.axis_index("c")
    s = lax.axis_index("s")
    w = s * NC + c
    pltpu.sync_copy(zflat3.at[pl.ds(s * NI * NSL, NI * NSL)], zv)
    pltpu.sync_copy(zv, agg_sh.at[pl.ds(s * NI * NSL, NI * NSL)])
    pltpu.sync_copy(t3f, tv)
    pltpu.sync_copy(src_f.at[pl.ds(w * EW, EW)], srci)
    pltpu.sync_copy(dst_f.at[pl.ds(w * EW, EW)], dsti)
    plsc.subcore_barrier()

    for i in range(NI):
        @pl.loop(0, NCH)
        def _(j):
            @pl.loop(0, ECH // 16)
            def _(k):
                sv = srci[pl.ds(j * ECH + k * 16, 16)] + i * NPAD
                vals[pl.ds(k * 16, 16)] = plsc.load_gather(tv, [sv])
                dsto[pl.ds(k * 16, 16)] = (
                    dsti[pl.ds(j * ECH + k * 16, 16)] + i * NPAD)

            pltpu.sync_copy(vals, agg_sh.at[dsto], add=True)

    plsc.subcore_barrier()
    pltpu.sync_copy(agg_sh.at[pl.ds(s * NI * NSL, NI * NSL)], zv)
    pltpu.sync_copy(
        zv, out_hbm.at[pl.ds(c * NI * NPAD + s * NI * NSL, NI * NSL)])


def _k2_body(degp_ref, x_ref, xs_ref):
    degp = degp_ref[...]
    deg = degp[0] + degp[1] + 1.0
    dinv = lax.rsqrt(deg)
    xs_ref[...] = x_ref[...] * dinv


def _k2_scale(degp, x_pad):
    B = 1024
    grid = NPAD // B
    return pl.pallas_call(
        _k2_body,
        grid=(grid,),
        in_specs=[
            pl.BlockSpec((NC, B, 1), lambda i: (0, i, 0)),
            pl.BlockSpec((B, D), lambda i: (i, 0)),
        ],
        out_specs=pl.BlockSpec((B, D), lambda i: (i, 0)),
        out_shape=jax.ShapeDtypeStruct((NPAD, D), jnp.float32),
    )(degp, x_pad)


def _k4_body(degp_ref, accp_ref, x_ref, wg_ref, bg_ref, wrr_ref,
             g_ref, t_ref, r_ref):
    degp = degp_ref[...]
    deg = degp[0] + degp[1] + 1.0
    dinv = lax.rsqrt(deg)
    accp = accp_ref[...]
    y = dinv * (accp[0] + accp[1]) + (dinv * dinv) * x_ref[...]
    wrr = wrr_ref[...]
    for i in range(NI):
        gi = jnp.maximum(
            jax.lax.dot_general(
                y, wg_ref[i], (((1,), (0,)), ((), ())),
                precision=lax.Precision.HIGHEST,
                preferred_element_type=jnp.float32) + bg_ref[i], 0.0)
        g_ref[i] = gi
        tr = jax.lax.dot_general(
            gi, wrr, (((1,), (0,)), ((), ())),
            precision=lax.Precision.HIGHEST,
            preferred_element_type=jnp.float32)
        t_ref[i] = tr[:, 0]
        r_ref[i] = tr[:, 1]


def _k4_layers(degp, accp, x_pad, W_gcn, b_gcn, Wrr):
    B = 1024
    grid = NPAD // B
    return pl.pallas_call(
        _k4_body,
        grid=(grid,),
        in_specs=[
            pl.BlockSpec((NC, B, 1), lambda i: (0, i, 0)),
            pl.BlockSpec((NC, B, D), lambda i: (0, i, 0)),
            pl.BlockSpec((B, D), lambda i: (i, 0)),
            pl.BlockSpec((NI, D, D), lambda i: (0, 0, 0)),
            pl.BlockSpec((NI, D), lambda i: (0, 0)),
            pl.BlockSpec((D, 2), lambda i: (0, 0)),
        ],
        out_specs=[
            pl.BlockSpec((NI, B, D), lambda i: (0, i, 0)),
            pl.BlockSpec((NI, B), lambda i: (0, i)),
            pl.BlockSpec((NI, B), lambda i: (0, i)),
        ],
        out_shape=[
            jax.ShapeDtypeStruct((NI, NPAD, D), jnp.float32),
            jax.ShapeDtypeStruct((NI, NPAD), jnp.float32),
            jax.ShapeDtypeStruct((NI, NPAD), jnp.float32),
        ],
    )(degp, accp, x_pad, W_gcn, b_gcn, Wrr)


def _k6_body(g_ref, aggp_ref, r_ref, batch_ref, wgo_ref, bgo_ref,
             a2_ref, ab_ref, batt_ref, out_ref):
    batch = batch_ref[...]
    gid = lax.broadcasted_iota(jnp.int32, (G, NPAD), 0)
    M = batch == gid
    Mf = M.astype(jnp.float32)
    aggp = aggp_ref[...]
    agg = aggp[0] + aggp[1]
    r3 = r_ref[...]
    batt = batt_ref[...]
    wgo = wgo_ref[...]
    bgo = bgo_ref[...]
    a2 = a2_ref[...]
    ab = ab_ref[...]

    neg = jnp.float32(-1e30)
    sc_cols = []
    for i in range(NI):
        xconv = (agg[i] + r3[i])[None, :] + batt
        m = jnp.max(jnp.where(M, xconv, neg), axis=1, keepdims=True)
        mx = jnp.max(jnp.where(M, m, neg), axis=0, keepdims=True)
        e = jnp.exp(jnp.minimum(xconv - mx, 0.0))
        P = Mf * e
        den = jnp.sum(P, axis=1, keepdims=True)
        S = P / (den + 1e-16)
        gx = jax.lax.dot_general(
            S, g_ref[i], (((1,), (0,)), ((), ())),
            precision=lax.Precision.HIGHEST,
            preferred_element_type=jnp.float32)
        gout = jnp.tanh(
            jax.lax.dot_general(
                gx, wgo, (((1,), (0,)), ((), ())),
                precision=lax.Precision.HIGHEST,
                preferred_element_type=jnp.float32) + bgo)
        sc_i = jnp.sum(gout * a2[:, i][None, :], axis=1, keepdims=True)
        sc_cols.append(sc_i + ab[0:1, i:i + 1])
    alls = jnp.concatenate(sc_cols, axis=1)
    mm = jnp.max(alls, axis=1, keepdims=True)
    ee = jnp.exp(alls - mm)
    alpha = ee / jnp.sum(ee, axis=1, keepdims=True)
    rep = jax.lax.dot_general(
        Mf, alpha, (((0,), (0,)), ((), ())),
        precision=lax.Precision.HIGHEST,
        preferred_element_type=jnp.float32)
    total = g_ref[0] * rep[:, 0][:, None]
    for i in range(1, NI):
        total = total + g_ref[i] * rep[:, i][:, None]
    out_ref[...] = total[0:N, :]


def _k6_final(g, aggp, r3, batch2, W_gout, b_gout, a2, ab, batt):
    return pl.pallas_call(
        _k6_body,
        out_shape=jax.ShapeDtypeStruct((N, D), jnp.float32),
    )(g, aggp, r3, batch2, W_gout, b_gout, a2, ab, batt)


def kernel(x, edge_index, batch, W_gcn, b_gcn, W_rel, W_root, b_att,
           W_gout, b_gout, a, a_bias):
    f32 = jnp.float32
    src = edge_index[0].astype(jnp.int32)
    dst = edge_index[1].astype(jnp.int32)
    pad_i = jnp.full((EPAD - E,), N, dtype=jnp.int32)
    src_p = jnp.concatenate([src, pad_i])
    dst_p = jnp.concatenate([dst, pad_i])
    src_r = src_p.reshape(EPAD // ECH, ECH)
    dst_r = dst_p.reshape(EPAD // ECH, ECH)
    src_r3 = src_p.reshape(EPAD // ECH3, ECH3)
    dst_r3 = dst_p.reshape(EPAD // ECH3, ECH3)

    x_pad = jnp.concatenate(
        [x.astype(f32), jnp.zeros((NPAD - N, D), f32)], axis=0)
    batch2 = jnp.concatenate(
        [batch.astype(jnp.int32), jnp.full((NPAD - N,), G, jnp.int32)]
    ).reshape(1, NPAD)

    ones128 = jnp.ones((ECH,), f32)
    zflat = jnp.zeros((NPAD,), f32)
    zrows = jnp.zeros((ECH3, D), f32)
    zflat3 = jnp.zeros((NI * NPAD,), f32)

    degf = _k1_deg(dst_r, ones128, zflat)
    degp = degf.reshape(NC, NPAD, 1)
    xs = _k2_scale(degp, x_pad)

    accf = _k3_edge(src_r3, dst_r3, xs, zrows)
    accp = accf.reshape(NC, NPAD, D)

    Wrr = jnp.concatenate([W_rel.astype(f32), W_root.astype(f32)], axis=1)
    g, t3, r3 = _k4_layers(degp, accp, x_pad, W_gcn.astype(f32),
                           b_gcn.astype(f32), Wrr)

    aggf = _k5_edge_t(src_p, dst_p, t3.reshape(NI * NPAD), zflat3)
    aggp = aggf.reshape(NC, NI, NPAD)

    a2 = a.astype(f32)[0]
    ab = a_bias.astype(f32)[0]
    batt = b_att.astype(f32).reshape(1, 1)
    out = _k6_final(g, aggp, r3, batch2, W_gout.astype(f32),
                    b_gout.astype(f32).reshape(1, D), a2, ab, batt)
    return out
